# Initial kernel scaffold; baseline (speedup 1.0000x reference)
import jax
import jax.numpy as jnp
from jax.experimental import pallas as pl


def kernel(x, edge_index, W1, bc1, g1, be1, W2, bc2, g2, be2):
    raise NotImplementedError("write your pallas kernel here")


# speedup vs baseline: 3.9723x; 3.9723x over previous
import functools

import jax
import jax.numpy as jnp
from jax import lax
from jax.experimental import pallas as pl
from jax.experimental.pallas import tpu as pltpu
from jax.experimental.pallas import tpu_sc as plsc

N = 10000
D_IN = 128
D_MID = 256
D_OUT = 128
K = 27
EPK = 12000
EPS = 1e-5

NC = 2
NS = 16
CHUNKS = 6
CB = 128
EPK_PAD = NS * CHUNKS * CB
NROW = 10240
ZPT = NROW // NS
K0 = 14

_mesh = plsc.VectorSubcoreMesh(
    core_axis_name="c", subcore_axis_name="s", num_cores=NC, num_subcores=NS)

_sc_scratch = [
    pltpu.VMEM((CHUNKS, CB), jnp.int32),
    pltpu.VMEM((CHUNKS, CB), jnp.int32),
    pltpu.VMEM((2, CB, D_IN), jnp.float32),
    pltpu.VMEM_SHARED((NROW, D_IN), jnp.float32),
    pltpu.SemaphoreType.DMA,
    pltpu.SemaphoreType.DMA,
]


def _gather_scatter_offset(table, sidx, didx, rows, acc, sems):
    ds = [None, None]
    ds[0] = pltpu.async_copy(table.at[sidx.at[0]], rows.at[0], sems[0])
    for j in range(CHUNKS):
        if j + 1 < CHUNKS:
            ds[(j + 1) % 2] = pltpu.async_copy(
                table.at[sidx.at[j + 1]], rows.at[(j + 1) % 2],
                sems[(j + 1) % 2])
        ds[j % 2].wait()
        pltpu.sync_copy(rows.at[j % 2], acc.at[didx.at[j]], add=True)


@functools.partial(
    pl.kernel,
    out_type=jax.ShapeDtypeStruct((K, NROW, D_IN), jnp.float32),
    mesh=_mesh,
    scratch_types=_sc_scratch,
)
def _sc_conv1_scatter(x_hbm, srcp_hbm, dstp_hbm, zeros_hbm, y_hbm,
                      sidx, didx, rows, acc, sem0, sem1):
    c = lax.axis_index("c")
    s = lax.axis_index("s")
    pltpu.sync_copy(zeros_hbm.at[pl.ds(s * ZPT, ZPT)],
                    acc.at[pl.ds(s * ZPT, ZPT)])
    plsc.subcore_barrier()
    k_lo = jnp.where(c == 0, 0, K0)
    k_n = jnp.where(c == 0, K0, K - K0)

    def body(i, carry):
        k = k_lo + i
        pltpu.sync_copy(srcp_hbm.at[k, s], sidx)
        pltpu.sync_copy(dstp_hbm.at[k, s], didx)
        _gather_scatter_offset(x_hbm, sidx, didx, rows, acc, (sem0, sem1))
        plsc.subcore_barrier()
        pltpu.sync_copy(acc.at[pl.ds(s * ZPT, ZPT)],
                        y_hbm.at[k, pl.ds(s * ZPT, ZPT)])
        plsc.subcore_barrier()
        return carry

    lax.fori_loop(0, k_n, body, 0)


@functools.partial(
    pl.kernel,
    out_type=jax.ShapeDtypeStruct((NC, NROW, D_OUT), jnp.float32),
    mesh=_mesh,
    scratch_types=_sc_scratch,
)
def _sc_conv2_scatter(z_hbm, srcp_hbm, dstp_hbm, zeros_hbm, out_hbm,
                      sidx, didx, rows, acc, sem0, sem1):
    c = lax.axis_index("c")
    s = lax.axis_index("s")
    pltpu.sync_copy(zeros_hbm.at[pl.ds(s * ZPT, ZPT)],
                    acc.at[pl.ds(s * ZPT, ZPT)])
    plsc.subcore_barrier()
    k_lo = jnp.where(c == 0, 0, K0)
    k_n = jnp.where(c == 0, K0, K - K0)

    def body(i, carry):
        k = k_lo + i
        pltpu.sync_copy(srcp_hbm.at[k, s], sidx)
        pltpu.sync_copy(dstp_hbm.at[k, s], didx)
        _gather_scatter_offset(z_hbm.at[k], sidx, didx, rows, acc,
                               (sem0, sem1))
        return carry

    lax.fori_loop(0, k_n, body, 0)
    plsc.subcore_barrier()
    pltpu.sync_copy(acc.at[pl.ds(s * ZPT, ZPT)],
                    out_hbm.at[c, pl.ds(s * ZPT, ZPT)])


def _tc_gemm1_body(y_ref, w_ref, b_ref, g_ref, be_ref, h_ref, acc_ref):
    k = pl.program_id(0)

    @pl.when(k == 0)
    def _init():
        acc_ref[...] = jnp.zeros_like(acc_ref)

    acc_ref[...] += jnp.dot(y_ref[0], w_ref[0],
                            preferred_element_type=jnp.float32)

    @pl.when(k == K - 1)
    def _fin():
        h = acc_ref[...] + b_ref[...]
        m = jnp.mean(h, axis=0, keepdims=True)
        hc = h - m
        v = jnp.mean(hc * hc, axis=0, keepdims=True)
        h = hc * lax.rsqrt(v + EPS) * g_ref[...] + be_ref[...]
        h_ref[...] = jnp.maximum(h, 0.0)


def _tc_gemm2_body(h_ref, w_ref, z_ref):
    z_ref[0] = jnp.dot(h_ref[...], w_ref[0],
                       preferred_element_type=jnp.float32)


def _tc_final_body(p_ref, x_ref, b_ref, g_ref, be_ref, o_ref):
    h = p_ref[0] + p_ref[1] + b_ref[...]
    m = jnp.mean(h, axis=0, keepdims=True)
    hc = h - m
    v = jnp.mean(hc * hc, axis=0, keepdims=True)
    h = hc * lax.rsqrt(v + EPS) * g_ref[...] + be_ref[...] + x_ref[...]
    o_ref[...] = jnp.maximum(h, 0.0)


def kernel(x, edge_index, W1, bc1, g1, be1, W2, bc2, g2, be2):
    src = edge_index[0].astype(jnp.int32).reshape(K, EPK)
    dst = edge_index[1].astype(jnp.int32).reshape(K, EPK)
    pad = EPK_PAD - EPK
    srcp = jnp.pad(src, ((0, 0), (0, pad))).reshape(K, NS, CHUNKS, CB)
    dstp = jnp.pad(dst, ((0, 0), (0, pad)),
                   constant_values=N).reshape(K, NS, CHUNKS, CB)
    zeros = jnp.zeros((NROW, D_IN), jnp.float32)

    W1eff = W1.at[0:K0 - 1].add(-W1[1:K0]).at[K0:K - 1].add(-W1[K0 + 1:K])

    y = _sc_conv1_scatter(x, srcp, dstp, zeros)

    h = pl.pallas_call(
        _tc_gemm1_body,
        grid=(K,),
        in_specs=[
            pl.BlockSpec((1, N, D_IN), lambda k: (k, 0, 0)),
            pl.BlockSpec((1, D_IN, D_MID), lambda k: (k, 0, 0)),
            pl.BlockSpec((1, D_MID), lambda k: (0, 0)),
            pl.BlockSpec((1, D_MID), lambda k: (0, 0)),
            pl.BlockSpec((1, D_MID), lambda k: (0, 0)),
        ],
        out_specs=pl.BlockSpec((N, D_MID), lambda k: (0, 0)),
        out_shape=jax.ShapeDtypeStruct((N, D_MID), jnp.float32),
        scratch_shapes=[pltpu.VMEM((N, D_MID), jnp.float32)],
    )(y, W1eff, bc1.reshape(1, -1), g1.reshape(1, -1), be1.reshape(1, -1))

    z = pl.pallas_call(
        _tc_gemm2_body,
        grid=(K,),
        in_specs=[
            pl.BlockSpec((N, D_MID), lambda k: (0, 0)),
            pl.BlockSpec((1, D_MID, D_OUT), lambda k: (k, 0, 0)),
        ],
        out_specs=pl.BlockSpec((1, N, D_OUT), lambda k: (k, 0, 0)),
        out_shape=jax.ShapeDtypeStruct((K, N, D_OUT), jnp.float32),
    )(h, W2)

    parts = _sc_conv2_scatter(z, srcp, dstp, zeros)

    out = pl.pallas_call(
        _tc_final_body,
        grid=(1,),
        in_specs=[
            pl.BlockSpec((NC, N, D_OUT), lambda i: (0, 0, 0)),
            pl.BlockSpec((N, D_OUT), lambda i: (0, 0)),
            pl.BlockSpec((1, D_OUT), lambda i: (0, 0)),
            pl.BlockSpec((1, D_OUT), lambda i: (0, 0)),
            pl.BlockSpec((1, D_OUT), lambda i: (0, 0)),
        ],
        out_specs=pl.BlockSpec((N, D_OUT), lambda i: (0, 0)),
        out_shape=jax.ShapeDtypeStruct((N, D_OUT), jnp.float32),
    )(parts, x, bc2.reshape(1, -1), g2.reshape(1, -1), be2.reshape(1, -1))

    return out


# speedup vs baseline: 4.0771x; 1.0264x over previous
import functools

import jax
import jax.numpy as jnp
from jax import lax
from jax.experimental import pallas as pl
from jax.experimental.pallas import tpu as pltpu
from jax.experimental.pallas import tpu_sc as plsc

N = 10000
D_IN = 128
D_MID = 256
D_OUT = 128
K = 27
EPK = 12000
EPS = 1e-5

NC = 2
NS = 16
CHUNKS = 12
CB = 64
NBUF = 4
LEAD = 2
EPK_PAD = NS * CHUNKS * CB
NROW = 10240
ZPT = NROW // NS
K0 = 14

_mesh = plsc.VectorSubcoreMesh(
    core_axis_name="c", subcore_axis_name="s", num_cores=NC, num_subcores=NS)

_sc_scratch = [
    pltpu.VMEM((2, CHUNKS, CB), jnp.int32),
    pltpu.VMEM((2, CHUNKS, CB), jnp.int32),
    pltpu.VMEM((NBUF, CB, D_IN), jnp.float32),
    pltpu.VMEM_SHARED((NROW, D_IN), jnp.float32),
    pltpu.SemaphoreType.DMA,
    pltpu.SemaphoreType.DMA,
    pltpu.SemaphoreType.DMA,
    pltpu.SemaphoreType.DMA,
    pltpu.SemaphoreType.DMA,
    pltpu.SemaphoreType.DMA,
    pltpu.SemaphoreType.DMA,
    pltpu.SemaphoreType.DMA,
    pltpu.SemaphoreType.DMA,
    pltpu.SemaphoreType.DMA,
]


def _fire_idx(srcp_hbm, dstp_hbm, k, s, sidx, didx, slot, isem):
    pltpu.async_copy(srcp_hbm.at[k, s], sidx.at[slot], isem)
    pltpu.async_copy(dstp_hbm.at[k, s], didx.at[slot], isem)


def _wait_idx(srcp_hbm, dstp_hbm, k, s, sidx, didx, slot, isem):
    pltpu.make_async_copy(srcp_hbm.at[k, s], sidx.at[slot], isem).wait()
    pltpu.make_async_copy(dstp_hbm.at[k, s], didx.at[slot], isem).wait()


def _gather_scatter_offset(table, sidx, didx, rows, acc, gsems, ssems,
                           mid=None):
    gd = [pltpu.async_copy(table.at[sidx.at[b]], rows.at[b], gsems[b])
          for b in range(NBUF)]
    if mid is not None:
        mid()
    sd = [None] * NBUF
    for j in range(CHUNKS):
        b = j % NBUF
        f = j + LEAD
        if NBUF <= f < CHUNKS:
            fb = f % NBUF
            sd[fb].wait()
            gd[fb] = pltpu.async_copy(table.at[sidx.at[f]], rows.at[fb],
                                      gsems[fb])
        gd[b].wait()
        sd[b] = pltpu.async_copy(rows.at[b], acc.at[didx.at[j]], ssems[b],
                                 add=True)
    for b in range(NBUF):
        sd[b].wait()


@functools.partial(
    pl.kernel,
    out_type=jax.ShapeDtypeStruct((K, NROW, D_IN), jnp.float32),
    mesh=_mesh,
    scratch_types=_sc_scratch,
)
def _sc_conv1_scatter(x_hbm, srcp_hbm, dstp_hbm, zeros_hbm, y_hbm,
                      sidx, didx, rows, acc, isem,
                      g0, g1, g2, g3, s0, s1, s2, s3, snap):
    gsems = (g0, g1, g2, g3)
    ssems = (s0, s1, s2, s3)
    c = lax.axis_index("c")
    s = lax.axis_index("s")
    k_lo = jnp.where(c == 0, 0, K0)
    k_n = jnp.where(c == 0, K0, K - K0)
    _fire_idx(srcp_hbm, dstp_hbm, k_lo, s, sidx, didx, 0, isem)
    pltpu.sync_copy(zeros_hbm.at[pl.ds(s * ZPT, ZPT)],
                    acc.at[pl.ds(s * ZPT, ZPT)])
    plsc.subcore_barrier()

    def body(i, carry):
        k = k_lo + i
        p = i % 2
        _wait_idx(srcp_hbm, dstp_hbm, k, s, sidx, didx, p, isem)

        @pl.when(i + 1 < k_n)
        def _prefetch():
            _fire_idx(srcp_hbm, dstp_hbm, k + 1, s, sidx, didx, 1 - p, isem)

        def _mid():
            @pl.when(i > 0)
            def _():
                pltpu.make_async_copy(
                    acc.at[pl.ds(s * ZPT, ZPT)],
                    y_hbm.at[k, pl.ds(s * ZPT, ZPT)], snap).wait()
            plsc.subcore_barrier()

        _gather_scatter_offset(x_hbm, sidx.at[p], didx.at[p], rows, acc,
                               gsems, ssems, mid=_mid)
        plsc.subcore_barrier()
        pltpu.async_copy(acc.at[pl.ds(s * ZPT, ZPT)],
                         y_hbm.at[k, pl.ds(s * ZPT, ZPT)], snap)
        return carry

    lax.fori_loop(0, k_n, body, 0)
    pltpu.make_async_copy(acc.at[pl.ds(s * ZPT, ZPT)],
                          y_hbm.at[k_lo, pl.ds(s * ZPT, ZPT)], snap).wait()


@functools.partial(
    pl.kernel,
    out_type=jax.ShapeDtypeStruct((NC, NROW, D_OUT), jnp.float32),
    mesh=_mesh,
    scratch_types=_sc_scratch,
)
def _sc_conv2_scatter(z_hbm, srcp_hbm, dstp_hbm, zeros_hbm, out_hbm,
                      sidx, didx, rows, acc, isem,
                      g0, g1, g2, g3, s0, s1, s2, s3, snap):
    gsems = (g0, g1, g2, g3)
    ssems = (s0, s1, s2, s3)
    c = lax.axis_index("c")
    s = lax.axis_index("s")
    k_lo = jnp.where(c == 0, 0, K0)
    k_n = jnp.where(c == 0, K0, K - K0)
    _fire_idx(srcp_hbm, dstp_hbm, k_lo, s, sidx, didx, 0, isem)
    pltpu.sync_copy(zeros_hbm.at[pl.ds(s * ZPT, ZPT)],
                    acc.at[pl.ds(s * ZPT, ZPT)])
    plsc.subcore_barrier()

    def body(i, carry):
        k = k_lo + i
        p = i % 2
        _wait_idx(srcp_hbm, dstp_hbm, k, s, sidx, didx, p, isem)

        @pl.when(i + 1 < k_n)
        def _prefetch():
            _fire_idx(srcp_hbm, dstp_hbm, k + 1, s, sidx, didx, 1 - p, isem)

        _gather_scatter_offset(z_hbm.at[k], sidx.at[p], didx.at[p], rows,
                               acc, gsems, ssems)
        return carry

    lax.fori_loop(0, k_n, body, 0)
    plsc.subcore_barrier()
    pltpu.sync_copy(acc.at[pl.ds(s * ZPT, ZPT)],
                    out_hbm.at[c, pl.ds(s * ZPT, ZPT)])


def _tc_gemm1_body(y_ref, w_ref, b_ref, g_ref, be_ref, h_ref, acc_ref):
    k = pl.program_id(0)

    @pl.when(k == 0)
    def _init():
        acc_ref[...] = jnp.zeros_like(acc_ref)

    acc_ref[...] += jnp.dot(y_ref[0], w_ref[0],
                            preferred_element_type=jnp.float32)

    @pl.when(k == K - 1)
    def _fin():
        h = acc_ref[...] + b_ref[...]
        m = jnp.mean(h, axis=0, keepdims=True)
        hc = h - m
        v = jnp.mean(hc * hc, axis=0, keepdims=True)
        h = hc * lax.rsqrt(v + EPS) * g_ref[...] + be_ref[...]
        h_ref[...] = jnp.maximum(h, 0.0)


def _tc_gemm2_body(h_ref, w_ref, z_ref):
    z_ref[0] = jnp.dot(h_ref[...], w_ref[0],
                       preferred_element_type=jnp.float32)


def _tc_final_body(p_ref, x_ref, b_ref, g_ref, be_ref, o_ref):
    h = p_ref[0] + p_ref[1] + b_ref[...]
    m = jnp.mean(h, axis=0, keepdims=True)
    hc = h - m
    v = jnp.mean(hc * hc, axis=0, keepdims=True)
    h = hc * lax.rsqrt(v + EPS) * g_ref[...] + be_ref[...] + x_ref[...]
    o_ref[...] = jnp.maximum(h, 0.0)


def kernel(x, edge_index, W1, bc1, g1, be1, W2, bc2, g2, be2):
    src = edge_index[0].astype(jnp.int32).reshape(K, EPK)
    dst = edge_index[1].astype(jnp.int32).reshape(K, EPK)
    pad = EPK_PAD - EPK
    srcp = jnp.pad(src, ((0, 0), (0, pad))).reshape(K, NS, CHUNKS, CB)
    dstp = jnp.pad(dst, ((0, 0), (0, pad)),
                   constant_values=N).reshape(K, NS, CHUNKS, CB)
    zeros = jnp.zeros((NROW, D_IN), jnp.float32)

    W1eff = W1.at[0:K0 - 1].add(-W1[1:K0]).at[K0:K - 1].add(-W1[K0 + 1:K])

    y = _sc_conv1_scatter(x, srcp, dstp, zeros)

    h = pl.pallas_call(
        _tc_gemm1_body,
        grid=(K,),
        in_specs=[
            pl.BlockSpec((1, N, D_IN), lambda k: (k, 0, 0)),
            pl.BlockSpec((1, D_IN, D_MID), lambda k: (k, 0, 0)),
            pl.BlockSpec((1, D_MID), lambda k: (0, 0)),
            pl.BlockSpec((1, D_MID), lambda k: (0, 0)),
            pl.BlockSpec((1, D_MID), lambda k: (0, 0)),
        ],
        out_specs=pl.BlockSpec((N, D_MID), lambda k: (0, 0)),
        out_shape=jax.ShapeDtypeStruct((N, D_MID), jnp.float32),
        scratch_shapes=[pltpu.VMEM((N, D_MID), jnp.float32)],
    )(y, W1eff, bc1.reshape(1, -1), g1.reshape(1, -1), be1.reshape(1, -1))

    z = pl.pallas_call(
        _tc_gemm2_body,
        grid=(K,),
        in_specs=[
            pl.BlockSpec((N, D_MID), lambda k: (0, 0)),
            pl.BlockSpec((1, D_MID, D_OUT), lambda k: (k, 0, 0)),
        ],
        out_specs=pl.BlockSpec((1, N, D_OUT), lambda k: (k, 0, 0)),
        out_shape=jax.ShapeDtypeStruct((K, N, D_OUT), jnp.float32),
    )(h, W2)

    parts = _sc_conv2_scatter(z, srcp, dstp, zeros)

    out = pl.pallas_call(
        _tc_final_body,
        grid=(1,),
        in_specs=[
            pl.BlockSpec((NC, N, D_OUT), lambda i: (0, 0, 0)),
            pl.BlockSpec((N, D_OUT), lambda i: (0, 0)),
            pl.BlockSpec((1, D_OUT), lambda i: (0, 0)),
            pl.BlockSpec((1, D_OUT), lambda i: (0, 0)),
            pl.BlockSpec((1, D_OUT), lambda i: (0, 0)),
        ],
        out_specs=pl.BlockSpec((N, D_OUT), lambda i: (0, 0)),
        out_shape=jax.ShapeDtypeStruct((N, D_OUT), jnp.float32),
    )(parts, x, bc2.reshape(1, -1), g2.reshape(1, -1), be2.reshape(1, -1))

    return out
